# baseline (speedup 1.0000x reference)
import jax
import jax.numpy as jnp
from jax import lax
from jax.experimental import pallas as pl
from jax.experimental.pallas import tpu as pltpu
from jax.experimental.pallas import tpu_sc as plsc

B = 16384
D = 64
NC, NS = 2, 16
NW = NC * NS
BPW = B // NW
CH = 128
NCH = BPW // CH
L = 16


def _body(ux_hbm, ix_hbm, uid2_hbm, iid2_hbm, out_hbm,
          idx_u, idx_i, tu_v, ti_v, bu0, bi0, bu1, bi1, out_v, sem):
    wid = lax.axis_index("s") * NC + lax.axis_index("c")
    base = wid * BPW

    for c in range(NCH):
        pltpu.sync_copy(ux_hbm.at[pl.ds(base + c * CH, CH)], idx_u.at[c])
        pltpu.sync_copy(ix_hbm.at[pl.ds(base + c * CH, CH)], idx_i.at[c])

    for c in range(NCH):
        for k in range(CH // L):
            sl = pl.ds(k * L, L)
            tu_v[c, sl] = lax.shift_right_logical(idx_u[c, sl], 1)
            ti_v[c, sl] = lax.shift_right_logical(idx_i[c, sl], 1)

    lanes = lax.iota(jnp.int32, L)
    perms = [lanes ^ sh for sh in (8, 4, 2, 1)]

    def fire(c, bu, bi):
        pltpu.async_copy(uid2_hbm.at[tu_v.at[c]], bu, sem)
        pltpu.async_copy(iid2_hbm.at[ti_v.at[c]], bi, sem)

    def drain(bu, bi):
        pltpu.make_async_copy(uid2_hbm.at[pl.ds(0, CH)], bu, sem).wait()
        pltpu.make_async_copy(uid2_hbm.at[pl.ds(0, CH)], bi, sem).wait()

    def compute(c, bu, bi):
        def grp(g, carry):
            r0 = g * L
            su16 = (idx_u[c, pl.ds(r0, L)] & 1) * 64
            si16 = (idx_i[c, pl.ds(r0, L)] & 1) * 64
            out16 = jnp.zeros((L,), jnp.float32)
            for jj in range(L):
                jc = r0 + jj
                su = su16[jj]
                si = si16[jj]
                acc = None
                for k in range(D // L):
                    u = bu[jc, pl.ds(su + k * L, L)]
                    v = bi[jc, pl.ds(si + k * L, L)]
                    p = u * v
                    acc = p if acc is None else acc + p
                for p_ in perms:
                    acc = acc + jnp.take_along_axis(
                        acc, p_, axis=0, mode="promise_in_bounds")
                out16 = jnp.where(lanes == jj, acc, out16)
            out_v[pl.ds(c * CH + r0, L)] = out16
            return carry

        lax.fori_loop(0, CH // L, grp, 0)

    fire(0, bu0, bi0)
    for i in range(NCH // 2):
        c0 = i * 2
        fire(c0 + 1, bu1, bi1)
        drain(bu0, bi0)
        compute(c0, bu0, bi0)
        if c0 + 2 < NCH:
            fire(c0 + 2, bu0, bi0)
        drain(bu1, bi1)
        compute(c0 + 1, bu1, bi1)
    pltpu.sync_copy(out_v, out_hbm.at[pl.ds(base, BPW)])


def kernel(x, uid_table, iid_table):
    ux = x[:, 0]
    ix = x[:, 1]
    uid2 = jnp.concatenate([uid_table[0::2], uid_table[1::2]], axis=1)
    iid2 = jnp.concatenate(
        [iid_table[0:100000:2], iid_table[1:100000:2]], axis=1)
    mesh = plsc.VectorSubcoreMesh(
        core_axis_name="c", subcore_axis_name="s",
        num_cores=NC, num_subcores=NS)
    run = pl.kernel(
        _body,
        out_type=jax.ShapeDtypeStruct((B,), jnp.float32),
        mesh=mesh,
        compiler_params=pltpu.CompilerParams(
            needs_layout_passes=False, use_tc_tiling_on_sc=True),
        scratch_types=[
            pltpu.VMEM((NCH, CH), jnp.int32),
            pltpu.VMEM((NCH, CH), jnp.int32),
            pltpu.VMEM((NCH, CH), jnp.int32),
            pltpu.VMEM((NCH, CH), jnp.int32),
            pltpu.VMEM((CH, 2 * D), jnp.float32),
            pltpu.VMEM((CH, 2 * D), jnp.float32),
            pltpu.VMEM((CH, 2 * D), jnp.float32),
            pltpu.VMEM((CH, 2 * D), jnp.float32),
            pltpu.VMEM((BPW,), jnp.float32),
            pltpu.SemaphoreType.DMA,
        ],
    )
    return run(ux, ix, uid2, iid2)


# speedup vs baseline: 12.9713x; 12.9713x over previous
import jax
import jax.numpy as jnp
from jax import lax
from jax.experimental import pallas as pl
from jax.experimental.pallas import tpu as pltpu
from jax.experimental.pallas import tpu_sc as plsc

B = 16384
D = 64
NC, NS = 2, 16
NW = NC * NS
BPW = B // NW
CH = 16
NCH = BPW // CH
L = 16
G = 8


def _body(ux_hbm, ix_hbm, uid_hbm, iid_hbm, out_hbm,
          uxv, ixv, bu0, bi0, bu1, bi1, out_v, sem):
    wid = lax.axis_index("s") * NC + lax.axis_index("c")
    base = wid * BPW

    pltpu.sync_copy(ux_hbm.at[pl.ds(base, BPW)], uxv)
    pltpu.sync_copy(ix_hbm.at[pl.ds(base, BPW)], ixv)

    lanes = lax.iota(jnp.int32, L)
    perms = [lanes ^ sh for sh in (8, 4, 2, 1)]
    u3 = uid_hbm.reshape(100000 // G, G, D)

    def fire(c, bu, bi):
        cb = c * CH
        for h in range(CH // L):
            tq = pl.ds(cb + h * L, L)
            tu16 = uxv[tq] & ~7
            ti16 = ixv[tq] & ~7
            for jj in range(L):
                jc = h * L + jj
                tu = pl.multiple_of(tu16[jj], G)
                ti = pl.multiple_of(ti16[jj], G)
                pltpu.async_copy(uid_hbm.at[pl.ds(tu, G)], bu.at[jc], sem)
                pltpu.async_copy(iid_hbm.at[pl.ds(ti, G)], bi.at[jc], sem)

    def drain(bu, bi):
        pltpu.make_async_copy(u3.at[pl.ds(0, CH)], bu, sem).wait()
        pltpu.make_async_copy(u3.at[pl.ds(0, CH)], bi, sem).wait()

    def compute(c, bu, bi):
        cb = c * CH
        for g in range(CH // L):
            r0 = g * L
            sq = pl.ds(cb + r0, L)
            su16 = uxv[sq] & 7
            si16 = ixv[sq] & 7
            out16 = jnp.zeros((L,), jnp.float32)
            for jj in range(L):
                jc = r0 + jj
                su = su16[jj]
                si = si16[jj]
                acc = None
                for k in range(D // L):
                    u = bu[jc, su, pl.ds(k * L, L)]
                    v = bi[jc, si, pl.ds(k * L, L)]
                    p = u * v
                    acc = p if acc is None else acc + p
                for p_ in perms:
                    acc = acc + jnp.take_along_axis(
                        acc, p_, axis=0, mode="promise_in_bounds")
                out16 = jnp.where(lanes == jj, acc, out16)
            out_v[pl.ds(cb + r0, L)] = out16

    fire(0, bu0, bi0)

    def pair(i, carry):
        c0 = i * 2
        c1 = c0 + 1
        fire(c1, bu1, bi1)
        drain(bu0, bi0)
        compute(c0, bu0, bi0)
        c2 = jnp.minimum(c0 + 2, NCH - 1)
        fire(c2, bu0, bi0)
        drain(bu1, bi1)
        compute(c1, bu1, bi1)
        return carry

    lax.fori_loop(0, NCH // 2, pair, 0)
    drain(bu0, bi0)
    pltpu.sync_copy(out_v, out_hbm.at[pl.ds(base, BPW)])


def kernel(x, uid_table, iid_table):
    ux = x[:, 0]
    ix = x[:, 1]
    mesh = plsc.VectorSubcoreMesh(
        core_axis_name="c", subcore_axis_name="s",
        num_cores=NC, num_subcores=NS)
    run = pl.kernel(
        _body,
        out_type=jax.ShapeDtypeStruct((B,), jnp.float32),
        mesh=mesh,
        compiler_params=pltpu.CompilerParams(
            needs_layout_passes=False, use_tc_tiling_on_sc=True),
        scratch_types=[
            pltpu.VMEM((BPW,), jnp.int32),
            pltpu.VMEM((BPW,), jnp.int32),
            pltpu.VMEM((CH, G, D), jnp.float32),
            pltpu.VMEM((CH, G, D), jnp.float32),
            pltpu.VMEM((CH, G, D), jnp.float32),
            pltpu.VMEM((CH, G, D), jnp.float32),
            pltpu.VMEM((BPW,), jnp.float32),
            pltpu.SemaphoreType.DMA,
        ],
    )
    return run(ux, ix, uid_table, iid_table)
